# baseline (speedup 1.0000x reference)
import functools

import jax
import jax.numpy as jnp
from jax import lax
from jax.experimental import pallas as pl
from jax.experimental.pallas import tpu as pltpu
from jax.experimental.pallas import tpu_sc as plsc

N, E, D, R, B = 10000, 320000, 128, 500, 512
NC, NS, L = 2, 16, 16
NW = NC * NS
H = D // 2
K = 64
CHUNKS = 320
HB = 80
NSB = CHUNKS // HB
EP = NS * K * CHUNKS
EWS = K * CHUNKS
NACC = 10240
SROW = NACC // NS
PBUF = 512
SEGW = B // NW
NBI = 10240
NB = 640
SB = NB // NS
RP = 512
DUMMY_DST = N + 8
NEG_INF = float("-inf")

_mesh = plsc.VectorSubcoreMesh(core_axis_name="c", subcore_axis_name="s")
_sc_params = pltpu.CompilerParams(use_tc_tiling_on_sc=False)
_sc_params_nl = pltpu.CompilerParams(use_tc_tiling_on_sc=False,
                                     needs_layout_passes=False)


def _edge_body(ent_hbm, rel_hbm, src_hbm, et_hbm, dst_hbm, bidx_hbm, zacc_hbm,
               zdeg_hbm, acc_out, deg_out, bcnt_out, src_blk, et_blk, dst_blk,
               e_bufs, r_bufs, p_bufs, ones_v, bidx_v, acc_sh, deg_sh,
               bcnt_sh, gsemA, gsemB, ssemA, ssemB):
  cid = lax.axis_index("c")
  sid = lax.axis_index("s")

  pltpu.sync_copy(zacc_hbm, acc_sh.at[pl.ds(sid * SROW, SROW)])

  @pl.when(cid == 0)
  def _():
    pltpu.sync_copy(zdeg_hbm, deg_sh.at[pl.ds(sid * SROW, SROW)])

  @pl.when(cid == 1)
  def _():
    pltpu.sync_copy(zdeg_hbm.at[pl.ds(0, SB)], bcnt_sh.at[pl.ds(sid * SB, SB)])

  @pl.loop(0, K)
  def _(r):
    ones_v[r] = jnp.ones((L,), jnp.float32)

  ent_off = cid * NACC
  rel_off = cid * RP

  plsc.subcore_barrier()

  def issue_gathers(ci, slot, gsem):
    pltpu.async_copy(ent_hbm.at[src_blk.at[ci]], e_bufs.at[slot], gsem)
    pltpu.async_copy(rel_hbm.at[et_blk.at[ci]], r_bufs.at[slot], gsem)

  def wait_gathers(slot, gsem):
    pltpu.make_async_copy(ent_hbm.at[src_blk.at[0]], e_bufs.at[slot],
                          gsem).wait()
    pltpu.make_async_copy(rel_hbm.at[et_blk.at[0]], r_bufs.at[slot],
                          gsem).wait()

  def multiply(slot):
    @plsc.parallel_loop(0, K, unroll=4)
    def _(r):
      for j in range(H // L):
        p_bufs[slot, r, pl.ds(j * L, L)] = (
            e_bufs[slot, r, pl.ds(j * L, L)] * r_bufs[slot, r, pl.ds(j * L, L)])

  def issue_scatter(ci, slot, ssem):
    pltpu.async_copy(p_bufs.at[slot], acc_sh.at[dst_blk.at[ci]], ssem,
                     add=True)

    @pl.when(cid == 0)
    def _():
      pltpu.async_copy(ones_v, deg_sh.at[dst_blk.at[ci]], ssem, add=True)

  def wait_scatter(ssem):
    pltpu.make_async_copy(p_bufs.at[0], acc_sh.at[dst_blk.at[0]], ssem).wait()

    @pl.when(cid == 0)
    def _():
      pltpu.make_async_copy(ones_v, deg_sh.at[dst_blk.at[0]], ssem).wait()

  for sb in range(NSB):
    rsl = pl.ds(sid * CHUNKS + sb * HB, HB)
    pltpu.sync_copy(src_hbm.at[rsl], src_blk)
    pltpu.sync_copy(et_hbm.at[rsl], et_blk)
    pltpu.sync_copy(dst_hbm.at[rsl], dst_blk)

    @pl.loop(0, HB)
    def _(r):
      for j in range(K // L):
        sl = (r, pl.ds(j * L, L))
        src_blk[sl] = src_blk[sl] + ent_off
        et_blk[sl] = et_blk[sl] + rel_off

    issue_gathers(0, 0, gsemA)

    @pl.loop(0, HB, step=2)
    def _(c):
      issue_gathers(c + 1, 1, gsemB)

      @pl.when(c > 0)
      def _():
        wait_scatter(ssemA)

      wait_gathers(0, gsemA)
      multiply(0)
      issue_scatter(c, 0, ssemA)

      @pl.when(c + 2 < HB)
      def _():
        issue_gathers(c + 2, 0, gsemA)

      @pl.when(c > 0)
      def _():
        wait_scatter(ssemB)

      wait_gathers(1, gsemB)
      multiply(1)
      issue_scatter(c + 1, 1, ssemB)

    wait_scatter(ssemA)
    wait_scatter(ssemB)

  @pl.when(cid == 1)
  def _():
    @pl.loop(0, NBI // NS // 64)
    def _(ci):
      bb = sid * (NBI // NS) + ci * 64
      pltpu.sync_copy(bidx_hbm.at[pl.ds(bb, 64)], bidx_v)
      pltpu.sync_copy(ones_v.at[pl.ds(0, 64)], bcnt_sh.at[bidx_v], add=True)

  plsc.subcore_barrier()

  sl = pl.ds(sid * SROW, SROW)

  @pl.when(cid == 0)
  def _():
    pltpu.sync_copy(acc_sh.at[sl], acc_out.at[0, sl])
    pltpu.sync_copy(deg_sh.at[sl], deg_out.at[sl])

  @pl.when(cid == 1)
  def _():
    pltpu.sync_copy(acc_sh.at[sl], acc_out.at[1, sl])
    slb = pl.ds(sid * SB, SB)
    pltpu.sync_copy(bcnt_sh.at[slb], bcnt_out.at[slb])


_edge_call = None


def _get_edge_pass():
  global _edge_call
  if _edge_call is None:
    _edge_call = pl.kernel(
        _edge_body, mesh=_mesh, compiler_params=_sc_params,
        out_type=[
            jax.ShapeDtypeStruct((NC, NACC, H), jnp.float32),
            jax.ShapeDtypeStruct((NACC, L), jnp.float32),
            jax.ShapeDtypeStruct((NB, L), jnp.float32),
        ],
        scratch_types=[
            pltpu.VMEM((HB, K), jnp.int32),
            pltpu.VMEM((HB, K), jnp.int32),
            pltpu.VMEM((HB, K), jnp.int32),
            pltpu.VMEM((2, K, H), jnp.float32),
            pltpu.VMEM((2, K, H), jnp.float32),
            pltpu.VMEM((2, K, H), jnp.float32),
            pltpu.VMEM((K, L), jnp.float32),
            pltpu.VMEM((64,), jnp.int32),
            pltpu.VMEM_SHARED((NACC, H), jnp.float32),
            pltpu.VMEM_SHARED((NACC, L), jnp.float32),
            pltpu.VMEM_SHARED((NB, L), jnp.float32),
            pltpu.SemaphoreType.DMA,
            pltpu.SemaphoreType.DMA,
            pltpu.SemaphoreType.DMA,
            pltpu.SemaphoreType.DMA,
        ],
    )
  return _edge_call


def _dense_body(lvl, relu, acc_ref, deg_ref, ent_ref, diam_ref, W_ref, b_ref,
                g_ref, bt_ref, out_ref):
  acc = jnp.concatenate([acc_ref[0], acc_ref[1]], axis=1)
  ent = jnp.concatenate([ent_ref[0], ent_ref[1]], axis=1)
  deg = deg_ref[:, :1]
  x = acc / jnp.maximum(deg, 1.0) + ent
  out = jnp.dot(x, W_ref[...], preferred_element_type=jnp.float32) + b_ref[...]
  core = out[:N]
  mean = jnp.sum(core, axis=0, keepdims=True) / N
  var = jnp.sum(core * core, axis=0, keepdims=True) / N - mean * mean
  o = g_ref[...] * (out - mean) * lax.rsqrt(var + 1e-5) + bt_ref[...]
  o = jnp.where(diam_ref[...] <= lvl, ent, o)
  if relu:
    o = jnp.maximum(o, 0.0)
  out_ref[0] = o[:, :H]
  out_ref[1] = o[:, H:]


def _dense(acc, deg, ent, diam, W, b, g, bt, lvl, relu):
  return pl.pallas_call(
      functools.partial(_dense_body, lvl, relu),
      out_shape=jax.ShapeDtypeStruct((NC, NACC, H), jnp.float32),
  )(acc, deg, ent, diam, W, b, g, bt)


def _relmm_body(rel_ref, w_ref, out_ref):
  rel = jnp.concatenate([rel_ref[0], rel_ref[1]], axis=1)
  o = jnp.dot(rel, w_ref[...], preferred_element_type=jnp.float32)
  out_ref[0] = o[:, :H]
  out_ref[1] = o[:, H:]


def _relmm(rel, W_rel):
  return pl.pallas_call(
      _relmm_body,
      out_shape=jax.ShapeDtypeStruct((NC, RP, H), jnp.float32),
  )(rel, W_rel)


def _pool_body(ent_hbm, bcnt_hbm, out_hbm, cnt_v, rows_v, macc_v, obuf_v,
               sem):
  cid = lax.axis_index("c")
  sid = lax.axis_index("s")
  wid = sid * NC + cid
  seg0 = wid * SEGW

  pltpu.sync_copy(bcnt_hbm, cnt_v)

  def pref_body(r, acc):
    return acc + cnt_v[r]

  base_vec = lax.fori_loop(0, seg0, pref_body, jnp.zeros((L,), jnp.float32))
  base0 = jnp.max(base_vec, axis=0).astype(jnp.int32)

  def span_body(r, acc):
    return acc + cnt_v[seg0 + r]

  span_vec = lax.fori_loop(0, SEGW, span_body, jnp.zeros((L,), jnp.float32))
  span = jnp.max(span_vec, axis=0).astype(jnp.int32)

  start = jnp.minimum(base0, NACC - PBUF)
  big = (base0 + span) <= (start + PBUF)

  def init_macc():
    for j in range(D // L):
      macc_v[pl.ds(j * L, L)] = jnp.full((L,), NEG_INF, jnp.float32)

  def flush_macc(i):
    for j in range(D // L):
      sl = pl.ds(j * L, L)
      v = macc_v[sl]
      obuf_v[i, sl] = jnp.where(v == NEG_INF, 0.0, v)

  @pl.when(big)
  def _():
    ga = pltpu.async_copy(ent_hbm.at[pl.ds(start, PBUF)], rows_v.at[0], sem)
    gb = pltpu.async_copy(ent_hbm.at[pl.ds(NACC + start, PBUF)], rows_v.at[1],
                          sem)
    ga.wait()
    gb.wait()
    base = base0
    for i in range(SEGW):
      cnt = jnp.max(cnt_v[seg0 + i], axis=0).astype(jnp.int32)
      end = base + cnt
      init_macc()

      def row_body(rr, carry):
        for j in range(H // L):
          sl = pl.ds(j * L, L)
          macc_v[sl] = jnp.maximum(macc_v[sl], rows_v[0, rr, sl])
          sr = pl.ds(H + j * L, L)
          macc_v[sr] = jnp.maximum(macc_v[sr], rows_v[1, rr, sl])
        return carry

      lax.fori_loop(base - start, end - start, row_body, 0)
      flush_macc(i)
      base = end

  @pl.when(jnp.logical_not(big))
  def _():
    base = base0
    for i in range(SEGW):
      cnt = jnp.max(cnt_v[seg0 + i], axis=0).astype(jnp.int32)
      end = base + cnt
      init_macc()

      def cond(r):
        return r < end

      def loop_body(r):
        cst = jnp.minimum(r, NACC - 256)
        off = r - cst
        ga = pltpu.async_copy(ent_hbm.at[pl.ds(cst, 256)],
                              rows_v.at[0, pl.ds(0, 256)], sem)
        gb = pltpu.async_copy(ent_hbm.at[pl.ds(NACC + cst, 256)],
                              rows_v.at[1, pl.ds(0, 256)], sem)
        ga.wait()
        gb.wait()

        def row_body(jr, carry):
          @pl.when(cst + jr < end)
          def _():
            for j in range(H // L):
              sl = pl.ds(j * L, L)
              macc_v[sl] = jnp.maximum(macc_v[sl], rows_v[0, jr, sl])
              sr = pl.ds(H + j * L, L)
              macc_v[sr] = jnp.maximum(macc_v[sr], rows_v[1, jr, sl])

          return carry

        lax.fori_loop(off, 256, row_body, 0)
        return cst + 256

      lax.while_loop(cond, loop_body, base)
      flush_macc(i)
      base = end

  pltpu.sync_copy(obuf_v, out_hbm.at[pl.ds(seg0, SEGW)])


def _pool(ent2, bcnt):
  return pl.kernel(
      _pool_body, mesh=_mesh, compiler_params=_sc_params_nl,
      out_type=jax.ShapeDtypeStruct((B, D), jnp.float32),
      scratch_types=[
          pltpu.VMEM((NB, L), jnp.float32),
          pltpu.VMEM((2, PBUF, H), jnp.float32),
          pltpu.VMEM((D,), jnp.float32),
          pltpu.VMEM((SEGW, D), jnp.float32),
          pltpu.SemaphoreType.DMA,
      ],
  )(ent2, bcnt)


def kernel(ent_embed, rel_embed, diameters, edge_index, edge_type, batch_idx,
           target_idx, W, W_rel, b, bn_gamma, bn_beta):
  i32 = jnp.int32
  src = edge_index[0].astype(i32)
  dst = edge_index[1].astype(i32)
  et = edge_type.astype(i32)
  pad = EP - E
  srcp = jnp.concatenate([src, jnp.zeros((pad,), i32)]).reshape(EP // K, K)
  dstp = jnp.concatenate([dst, jnp.full((pad,), DUMMY_DST,
                                        i32)]).reshape(EP // K, K)
  etp = jnp.concatenate([et, jnp.zeros((pad,), i32)]).reshape(EP // K, K)
  bidxp = jnp.concatenate(
      [batch_idx.astype(i32), jnp.full((NBI - N,), B, i32)])
  entp = jnp.pad(ent_embed, ((0, NACC - N), (0, 0)))
  ent_stack = jnp.stack([entp[:, :H], entp[:, H:]])
  relp = jnp.pad(rel_embed, ((0, RP - R), (0, 0)))
  rel_stack = jnp.stack([relp[:, :H], relp[:, H:]])
  diamp = jnp.pad(diameters.astype(i32), (0, NACC - N)).reshape(NACC, 1)
  zacc = jnp.zeros((SROW, H), jnp.float32)
  zdeg = jnp.zeros((SROW, L), jnp.float32)
  b2 = b.reshape(1, D)
  g2 = bn_gamma.reshape(1, D)
  bt2 = bn_beta.reshape(1, D)

  edge_pass = _get_edge_pass()
  ent_flat = ent_stack.reshape(NC * NACC, H)
  rel_flat = rel_stack.reshape(NC * RP, H)

  acc0, deg0, _ = edge_pass(ent_flat, rel_flat, srcp, etp, dstp, bidxp, zacc,
                            zdeg)
  ent1 = _dense(acc0, deg0, ent_stack, diamp, W, b2, g2, bt2, 0, True)
  rel1 = _relmm(rel_stack, W_rel)

  acc1, deg1, bcnt = edge_pass(ent1.reshape(NC * NACC, H),
                               rel1.reshape(NC * RP, H), srcp, etp, dstp,
                               bidxp, zacc, zdeg)
  ent2 = _dense(acc1, deg1, ent1, diamp, W, b2, g2, bt2, 1, False)

  return _pool(ent2.reshape(NC * NACC, H), bcnt)


# speedup vs baseline: 1.6431x; 1.6431x over previous
import functools

import jax
import jax.numpy as jnp
import numpy as np
from jax import lax
from jax.experimental import pallas as pl
from jax.experimental.pallas import tpu as pltpu
from jax.experimental.pallas import tpu_sc as plsc

N, E, D, R, B = 10000, 320000, 128, 500, 512
NC, NS, L = 2, 16, 16
NW = NC * NS
H = D // 2
K = 128
CHUNKS = 160
HB = 40
NSB = CHUNKS // HB
EP = NS * K * CHUNKS
EWS = K * CHUNKS
NACC = 10240
SROW = NACC // NS
PBUF = 512
SEGW = B // NW
NBI = 10240
NB = 640
SB = NB // NS
RP = 512
DUMMY_DST = N + 8
NEG_INF = float("-inf")

_mesh = plsc.VectorSubcoreMesh(core_axis_name="c", subcore_axis_name="s")
_sc_params = pltpu.CompilerParams(use_tc_tiling_on_sc=False)
_sc_params_nl = pltpu.CompilerParams(use_tc_tiling_on_sc=False,
                                     needs_layout_passes=False)


def _edge_body(ent_hbm, rel_hbm, src_hbm, et_hbm, dst_hbm, bidx_hbm, zacc_hbm,
               zdeg_hbm, acc_out, deg_out, bcnt_out, src_blk, et_blk, dst_blk,
               e_bufs, r_bufs, p_bufs, ones_v, bidx_v, deg_v, acc_sh,
               bcnt_sh, gsemA, gsemB, ssemA, ssemB):
  cid = lax.axis_index("c")
  sid = lax.axis_index("s")

  pltpu.sync_copy(zacc_hbm, acc_sh.at[pl.ds(sid * SROW, SROW)])

  @pl.when(cid == 1)
  def _():
    pltpu.sync_copy(zdeg_hbm.at[pl.ds(0, SB)], bcnt_sh.at[pl.ds(sid * SB, SB)])

  @pl.loop(0, NACC // L)
  def _(r):
    deg_v[pl.ds(r * L, L)] = jnp.zeros((L,), jnp.float32)

  @pl.loop(0, 64)
  def _(r):
    ones_v[r] = jnp.ones((L,), jnp.float32)

  ent_off = cid * NACC
  rel_off = cid * RP

  plsc.subcore_barrier()

  def issue_gathers(ci, slot, gsem):
    pltpu.async_copy(ent_hbm.at[src_blk.at[ci]], e_bufs.at[slot], gsem)
    pltpu.async_copy(rel_hbm.at[et_blk.at[ci]], r_bufs.at[slot], gsem)

  def wait_gathers(slot, gsem):
    pltpu.make_async_copy(ent_hbm.at[src_blk.at[0]], e_bufs.at[slot],
                          gsem).wait()
    pltpu.make_async_copy(rel_hbm.at[et_blk.at[0]], r_bufs.at[slot],
                          gsem).wait()

  ones16 = jnp.ones((L,), jnp.float32)

  def multiply(ci, slot):
    @plsc.parallel_loop(0, K, unroll=4)
    def _(r):
      for j in range(H // (2 * L)):
        sl = pl.ds(j * 2 * L, 2 * L)
        ea, eb = plsc.unpack(e_bufs[slot, r, sl],
                             format=plsc.PackFormat.INTERLEAVED)
        ra, rb = plsc.unpack(r_bufs[slot, r, sl],
                             format=plsc.PackFormat.INTERLEAVED)
        p_bufs[slot, r, pl.ds(j * 2 * L, L)] = ea * ra
        p_bufs[slot, r, pl.ds(j * 2 * L + L, L)] = eb * rb

    @pl.when(cid == 0)
    def _():
      @pl.loop(0, K // L)
      def _(r):
        idx = dst_blk[ci, pl.ds(r * L, L)]
        plsc.addupdate_scatter(deg_v, [idx], ones16)

  def issue_scatter(ci, slot, ssem):
    pltpu.async_copy(p_bufs.at[slot], acc_sh.at[dst_blk.at[ci]], ssem,
                     add=True)

  def wait_scatter(ssem):
    pltpu.make_async_copy(p_bufs.at[0], acc_sh.at[dst_blk.at[0]], ssem).wait()

  for sb in range(NSB):
    rsl = pl.ds(sid * CHUNKS + sb * HB, HB)
    pltpu.sync_copy(src_hbm.at[rsl], src_blk)
    pltpu.sync_copy(et_hbm.at[rsl], et_blk)
    pltpu.sync_copy(dst_hbm.at[rsl], dst_blk)

    @pl.loop(0, HB)
    def _(r):
      for j in range(K // L):
        sl = (r, pl.ds(j * L, L))
        src_blk[sl] = src_blk[sl] + ent_off
        et_blk[sl] = et_blk[sl] + rel_off

    issue_gathers(0, 0, gsemA)

    @pl.loop(0, HB, step=2)
    def _(c):
      issue_gathers(c + 1, 1, gsemB)

      @pl.when(c > 0)
      def _():
        wait_scatter(ssemA)

      wait_gathers(0, gsemA)
      multiply(c, 0)
      issue_scatter(c, 0, ssemA)

      @pl.when(c + 2 < HB)
      def _():
        issue_gathers(c + 2, 0, gsemA)

      @pl.when(c > 0)
      def _():
        wait_scatter(ssemB)

      wait_gathers(1, gsemB)
      multiply(c + 1, 1)
      issue_scatter(c + 1, 1, ssemB)

    wait_scatter(ssemA)
    wait_scatter(ssemB)

  @pl.when(cid == 1)
  def _():
    @pl.loop(0, NBI // NS // 64)
    def _(ci):
      bb = sid * (NBI // NS) + ci * 64
      pltpu.sync_copy(bidx_hbm.at[pl.ds(bb, 64)], bidx_v)
      pltpu.sync_copy(ones_v.at[pl.ds(0, 64)], bcnt_sh.at[bidx_v], add=True)

  plsc.subcore_barrier()

  sl = pl.ds(sid * SROW, SROW)

  @pl.when(cid == 0)
  def _():
    pltpu.sync_copy(acc_sh.at[sl], acc_out.at[0, sl])
    pltpu.sync_copy(deg_v, deg_out.at[sid])

  @pl.when(cid == 1)
  def _():
    pltpu.sync_copy(acc_sh.at[sl], acc_out.at[1, sl])
    slb = pl.ds(sid * SB, SB)
    pltpu.sync_copy(bcnt_sh.at[slb], bcnt_out.at[slb])


_edge_call = None


def _get_edge_pass():
  global _edge_call
  if _edge_call is None:
    _edge_call = pl.kernel(
        _edge_body, mesh=_mesh, compiler_params=_sc_params_nl,
        out_type=[
            jax.ShapeDtypeStruct((NC, NACC, H), jnp.float32),
            jax.ShapeDtypeStruct((NS, NACC), jnp.float32),
            jax.ShapeDtypeStruct((NB, L), jnp.float32),
        ],
        scratch_types=[
            pltpu.VMEM((HB, K), jnp.int32),
            pltpu.VMEM((HB, K), jnp.int32),
            pltpu.VMEM((HB, K), jnp.int32),
            pltpu.VMEM((2, K, H), jnp.bfloat16),
            pltpu.VMEM((2, K, H), jnp.bfloat16),
            pltpu.VMEM((2, K, H), jnp.float32),
            pltpu.VMEM((64, L), jnp.float32),
            pltpu.VMEM((64,), jnp.int32),
            pltpu.VMEM((NACC,), jnp.float32),
            pltpu.VMEM_SHARED((NACC, H), jnp.float32),
            pltpu.VMEM_SHARED((NB, L), jnp.float32),
            pltpu.SemaphoreType.DMA,
            pltpu.SemaphoreType.DMA,
            pltpu.SemaphoreType.DMA,
            pltpu.SemaphoreType.DMA,
        ],
    )
  return _edge_call


def _unpack_perm():
  perm = np.zeros((D, D), np.float32)
  for i in range(D):
    g, s = divmod(i, 2 * L)
    nat = 2 * L * g + (2 * s if s < L else 2 * (s - L) + 1)
    perm[i, nat] = 1.0
  return perm


_PERM = _unpack_perm()


def _dense_body(lvl, relu, acc_ref, deg_ref, ent_ref, diam_ref, W_ref, b_ref,
                g_ref, bt_ref, perm_ref, out_ref, outb_ref):
  acc_stored = jnp.concatenate([acc_ref[0], acc_ref[1]], axis=1)
  acc = jnp.dot(acc_stored, perm_ref[...], preferred_element_type=jnp.float32)
  ent = jnp.concatenate([ent_ref[0], ent_ref[1]], axis=1)
  deg = jnp.sum(deg_ref[...], axis=0)[:, None]
  x = acc / jnp.maximum(deg, 1.0) + ent
  out = jnp.dot(x, W_ref[...], preferred_element_type=jnp.float32) + b_ref[...]
  core = out[:N]
  mean = jnp.sum(core, axis=0, keepdims=True) / N
  var = jnp.sum(core * core, axis=0, keepdims=True) / N - mean * mean
  o = g_ref[...] * (out - mean) * lax.rsqrt(var + 1e-5) + bt_ref[...]
  o = jnp.where(diam_ref[...] <= lvl, ent, o)
  if relu:
    o = jnp.maximum(o, 0.0)
  out_ref[0] = o[:, :H]
  out_ref[1] = o[:, H:]
  ob = o.astype(jnp.bfloat16)
  outb_ref[0] = ob[:, :H]
  outb_ref[1] = ob[:, H:]


def _dense(acc, deg, ent, diam, W, b, g, bt, lvl, relu):
  return pl.pallas_call(
      functools.partial(_dense_body, lvl, relu),
      out_shape=[
          jax.ShapeDtypeStruct((NC, NACC, H), jnp.float32),
          jax.ShapeDtypeStruct((NC, NACC, H), jnp.bfloat16),
      ],
  )(acc, deg, ent, diam, W, b, g, bt, jnp.asarray(_PERM))


def _relmm_body(rel_ref, w_ref, out_ref):
  rel = jnp.concatenate([rel_ref[0], rel_ref[1]], axis=1)
  o = jnp.dot(rel, w_ref[...],
              preferred_element_type=jnp.float32).astype(jnp.bfloat16)
  out_ref[0] = o[:, :H]
  out_ref[1] = o[:, H:]


def _relmm(rel, W_rel):
  return pl.pallas_call(
      _relmm_body,
      out_shape=jax.ShapeDtypeStruct((NC, RP, H), jnp.bfloat16),
  )(rel, W_rel)


def _pool_body(ent_hbm, bcnt_hbm, out_hbm, cnt_v, rows_v, macc_v, obuf_v,
               sem):
  cid = lax.axis_index("c")
  sid = lax.axis_index("s")
  wid = sid * NC + cid
  seg0 = wid * SEGW

  pltpu.sync_copy(bcnt_hbm, cnt_v)

  def pref_body(r, acc):
    return acc + cnt_v[r]

  base_vec = lax.fori_loop(0, seg0, pref_body, jnp.zeros((L,), jnp.float32))
  base0 = jnp.max(base_vec, axis=0).astype(jnp.int32)

  def span_body(r, acc):
    return acc + cnt_v[seg0 + r]

  span_vec = lax.fori_loop(0, SEGW, span_body, jnp.zeros((L,), jnp.float32))
  span = jnp.max(span_vec, axis=0).astype(jnp.int32)

  start = jnp.minimum(base0, NACC - PBUF)
  big = (base0 + span) <= (start + PBUF)

  def init_macc():
    for j in range(D // L):
      macc_v[pl.ds(j * L, L)] = jnp.full((L,), NEG_INF, jnp.float32)

  def flush_macc(i):
    for j in range(D // L):
      sl = pl.ds(j * L, L)
      v = macc_v[sl]
      obuf_v[i, sl] = jnp.where(v == NEG_INF, 0.0, v)

  @pl.when(big)
  def _():
    ga = pltpu.async_copy(ent_hbm.at[pl.ds(start, PBUF)], rows_v.at[0], sem)
    gb = pltpu.async_copy(ent_hbm.at[pl.ds(NACC + start, PBUF)], rows_v.at[1],
                          sem)
    ga.wait()
    gb.wait()
    base = base0
    for i in range(SEGW):
      cnt = jnp.max(cnt_v[seg0 + i], axis=0).astype(jnp.int32)
      end = base + cnt
      init_macc()

      def row_body(rr, carry):
        for j in range(H // L):
          sl = pl.ds(j * L, L)
          macc_v[sl] = jnp.maximum(macc_v[sl], rows_v[0, rr, sl])
          sr = pl.ds(H + j * L, L)
          macc_v[sr] = jnp.maximum(macc_v[sr], rows_v[1, rr, sl])
        return carry

      lax.fori_loop(base - start, end - start, row_body, 0)
      flush_macc(i)
      base = end

  @pl.when(jnp.logical_not(big))
  def _():
    base = base0
    for i in range(SEGW):
      cnt = jnp.max(cnt_v[seg0 + i], axis=0).astype(jnp.int32)
      end = base + cnt
      init_macc()

      def cond(r):
        return r < end

      def loop_body(r):
        cst = jnp.minimum(r, NACC - 256)
        off = r - cst
        ga = pltpu.async_copy(ent_hbm.at[pl.ds(cst, 256)],
                              rows_v.at[0, pl.ds(0, 256)], sem)
        gb = pltpu.async_copy(ent_hbm.at[pl.ds(NACC + cst, 256)],
                              rows_v.at[1, pl.ds(0, 256)], sem)
        ga.wait()
        gb.wait()

        def row_body(jr, carry):
          @pl.when(cst + jr < end)
          def _():
            for j in range(H // L):
              sl = pl.ds(j * L, L)
              macc_v[sl] = jnp.maximum(macc_v[sl], rows_v[0, jr, sl])
              sr = pl.ds(H + j * L, L)
              macc_v[sr] = jnp.maximum(macc_v[sr], rows_v[1, jr, sl])

          return carry

        lax.fori_loop(off, 256, row_body, 0)
        return cst + 256

      lax.while_loop(cond, loop_body, base)
      flush_macc(i)
      base = end

  pltpu.sync_copy(obuf_v, out_hbm.at[pl.ds(seg0, SEGW)])


def _pool(ent2, bcnt):
  return pl.kernel(
      _pool_body, mesh=_mesh, compiler_params=_sc_params_nl,
      out_type=jax.ShapeDtypeStruct((B, D), jnp.float32),
      scratch_types=[
          pltpu.VMEM((NB, L), jnp.float32),
          pltpu.VMEM((2, PBUF, H), jnp.float32),
          pltpu.VMEM((D,), jnp.float32),
          pltpu.VMEM((SEGW, D), jnp.float32),
          pltpu.SemaphoreType.DMA,
      ],
  )(ent2, bcnt)


def kernel(ent_embed, rel_embed, diameters, edge_index, edge_type, batch_idx,
           target_idx, W, W_rel, b, bn_gamma, bn_beta):
  i32 = jnp.int32
  src = edge_index[0].astype(i32)
  dst = edge_index[1].astype(i32)
  et = edge_type.astype(i32)
  pad = EP - E
  srcp = jnp.concatenate([src, jnp.zeros((pad,), i32)]).reshape(EP // K, K)
  dstp = jnp.concatenate([dst, jnp.full((pad,), DUMMY_DST,
                                        i32)]).reshape(EP // K, K)
  etp = jnp.concatenate([et, jnp.zeros((pad,), i32)]).reshape(EP // K, K)
  bidxp = jnp.concatenate(
      [batch_idx.astype(i32), jnp.full((NBI - N,), B, i32)])
  entp = jnp.pad(ent_embed, ((0, NACC - N), (0, 0)))
  ent_stack = jnp.stack([entp[:, :H], entp[:, H:]])
  relp = jnp.pad(rel_embed, ((0, RP - R), (0, 0)))
  rel_stack = jnp.stack([relp[:, :H], relp[:, H:]])
  diamp = jnp.pad(diameters.astype(i32), (0, NACC - N)).reshape(NACC, 1)
  zacc = jnp.zeros((SROW, H), jnp.float32)
  zdeg = jnp.zeros((SROW, L), jnp.float32)
  b2 = b.reshape(1, D)
  g2 = bn_gamma.reshape(1, D)
  bt2 = bn_beta.reshape(1, D)

  edge_pass = _get_edge_pass()
  ent_flat = ent_stack.astype(jnp.bfloat16).reshape(NC * NACC, H)
  rel_flat = rel_stack.astype(jnp.bfloat16).reshape(NC * RP, H)

  acc0, deg0, _ = edge_pass(ent_flat, rel_flat, srcp, etp, dstp, bidxp, zacc,
                            zdeg)
  ent1, ent1b = _dense(acc0, deg0, ent_stack, diamp, W, b2, g2, bt2, 0, True)
  rel1 = _relmm(rel_stack, W_rel)

  acc1, deg1, bcnt = edge_pass(ent1b.reshape(NC * NACC, H),
                               rel1.reshape(NC * RP, H), srcp, etp, dstp,
                               bidxp, zacc, zdeg)
  ent2, _ = _dense(acc1, deg1, ent1, diamp, W, b2, g2, bt2, 1, False)

  return _pool(ent2.reshape(NC * NACC, H), bcnt)
